# Initial kernel scaffold; baseline (speedup 1.0000x reference)
import jax
import jax.numpy as jnp
from jax.experimental import pallas as pl


def kernel(x, table):
    raise NotImplementedError("write your pallas kernel here")


# speedup vs baseline: 1.4426x; 1.4426x over previous
import functools

import jax
import jax.numpy as jnp
from jax import lax
from jax.experimental import pallas as pl
from jax.experimental.pallas import tpu as pltpu
from jax.experimental.pallas import tpu_sc as plsc

_D = 1024
_NC = 2
_NS = 16
_NW = _NC * _NS
_K = 32


def _make_gather(n_tokens: int):
    bpw = n_tokens // _NW
    nchunk = bpw // _K

    @functools.partial(
        pl.kernel,
        out_type=jax.ShapeDtypeStruct((n_tokens, _D), jnp.float32),
        mesh=plsc.VectorSubcoreMesh(core_axis_name="c", subcore_axis_name="s"),
        scratch_types=[
            pltpu.VMEM((nchunk, _K), jnp.int32),
            pltpu.VMEM((_K, _D), jnp.float32),
            pltpu.SemaphoreType.DMA,
        ],
    )
    def gather_kernel(idx_hbm, table_hbm, out_hbm, idx_v, rows_v, gsem):
        wid = lax.axis_index("s") * _NC + lax.axis_index("c")
        pltpu.sync_copy(idx_hbm.at[wid], idx_v)
        base = wid * bpw

        def step(j, carry):
            pltpu.async_copy(table_hbm.at[idx_v.at[j]], rows_v, gsem).wait()
            pltpu.sync_copy(rows_v, out_hbm.at[pl.ds(base + j * _K, _K)])
            return carry

        lax.fori_loop(0, nchunk, step, 0)

    return gather_kernel


def kernel(x, table):
    batch, seq = x.shape
    n = batch * seq
    idx = x.reshape(_NW, n // (_NW * _K), _K).astype(jnp.int32)
    out = _make_gather(n)(idx, table)
    return out.reshape(batch, seq, _D)


# speedup vs baseline: 1.5686x; 1.0874x over previous
import functools

import jax
import jax.numpy as jnp
from jax import lax
from jax.experimental import pallas as pl
from jax.experimental.pallas import tpu as pltpu
from jax.experimental.pallas import tpu_sc as plsc

_D = 1024
_NC = 2
_NS = 16
_NW = _NC * _NS
_K = 32


def _make_gather(n_tokens: int):
    bpw = n_tokens // _NW
    nchunk = bpw // _K

    @functools.partial(
        pl.kernel,
        out_type=jax.ShapeDtypeStruct((n_tokens, _D), jnp.float32),
        mesh=plsc.VectorSubcoreMesh(core_axis_name="c", subcore_axis_name="s"),
        scratch_types=[
            pltpu.VMEM((nchunk, _K), jnp.int32),
            pltpu.VMEM((_K, _D), jnp.float32),
            pltpu.VMEM((_K, _D), jnp.float32),
            pltpu.SemaphoreType.DMA,
            pltpu.SemaphoreType.DMA,
            pltpu.SemaphoreType.DMA,
            pltpu.SemaphoreType.DMA,
        ],
    )
    def gather_kernel(idx_hbm, table_hbm, out_hbm, idx_v, rows0, rows1,
                      g0, g1, o0, o1):
        wid = lax.axis_index("s") * _NC + lax.axis_index("c")
        pltpu.sync_copy(idx_hbm.at[wid], idx_v)
        base = wid * bpw
        rows = (rows0, rows1)
        gs = (g0, g1)
        os_ = (o0, o1)

        for b in range(2):
            pltpu.async_copy(table_hbm.at[idx_v.at[b]], rows[b], gs[b])

        def step(i, carry):
            for b in range(2):
                j = i * 2 + b
                pltpu.make_async_copy(
                    table_hbm.at[idx_v.at[0]], rows[b], gs[b]).wait()
                pltpu.async_copy(
                    rows[b], out_hbm.at[pl.ds(base + j * _K, _K)], os_[b])
            for b in range(2):
                j2 = i * 2 + b + 2

                @pl.when(j2 < nchunk)
                def _():
                    pltpu.make_async_copy(
                        rows[b], out_hbm.at[pl.ds(base, _K)], os_[b]).wait()
                    pltpu.async_copy(table_hbm.at[idx_v.at[j2]], rows[b], gs[b])
            return carry

        lax.fori_loop(0, nchunk // 2, step, 0)
        for b in range(2):
            pltpu.make_async_copy(
                rows[b], out_hbm.at[pl.ds(base, _K)], os_[b]).wait()

    return gather_kernel


def kernel(x, table):
    batch, seq = x.shape
    n = batch * seq
    idx = x.reshape(_NW, n // (_NW * _K), _K).astype(jnp.int32)
    out = _make_gather(n)(idx, table)
    return out.reshape(batch, seq, _D)
